# baseline (speedup 1.0000x reference)
import functools

import jax
import jax.numpy as jnp
from jax import lax
from jax.experimental import pallas as pl
from jax.experimental.pallas import tpu as pltpu
from jax.experimental.pallas import tpu_sc as plsc

N = 10000
E = 320000
D = 128
DE = 4
NC = 1
NG = 8

SC_CORES = 2
SC_TILES = 16
NW = SC_CORES * SC_TILES
CHUNK = 64
NL = 160
NCHUNK = NW * NL
E_PAD = NCHUNK * CHUNK
N_PAD = 10240
ROWS_PER_TILE = N_PAD // SC_TILES


def _xm_body(x_ref, w1_ref, b_ref, o_ref):
    o_ref[...] = (
        jnp.dot(x_ref[...], w1_ref[...], preferred_element_type=jnp.float32)
        + b_ref[...]
    )


def _em_body(ea_ref, w2_ref, o_ref):
    o_ref[...] = lax.dot_general(
        ea_ref[...], w2_ref[...], (((1,), (0,)), ((), ())),
        preferred_element_type=jnp.float32,
    )


def _sc_agg_body(xm_hbm, em_hbm, sd_hbm, out_hbm,
                 idx0, idx1, rows0, rows1, em0, em1, agg_sh,
                 sg0, sg1, se0, se1, ss0, ss1):
    cid = lax.axis_index("c")
    sid = lax.axis_index("s")
    wid = sid * SC_CORES + cid
    idx = (idx0, idx1)
    rows = (rows0, rows1)
    emv = (em0, em1)
    sg = (sg0, sg1)
    se = (se0, se1)
    ss = (ss0, ss1)

    zvec = jnp.zeros((16,), jnp.float32)

    def zero_body(i, _):
        r = i // (D // 16)
        j = i % (D // 16)
        rows0[r, pl.ds(j * 16, 16)] = zvec
        return 0

    lax.fori_loop(0, CHUNK * (D // 16), zero_body, 0)
    for i in range(ROWS_PER_TILE // CHUNK):
        pltpu.sync_copy(
            rows0, agg_sh.at[pl.ds(sid * ROWS_PER_TILE + i * CHUNK, CHUNK)]
        )
    plsc.subcore_barrier()

    def issue(i_chunk, b):
        c = i_chunk * NW + wid
        pltpu.sync_copy(sd_hbm.at[c], idx[b])
        pltpu.async_copy(xm_hbm.at[idx[b].at[0]], rows[b], sg[b])
        pltpu.async_copy(em_hbm.at[pl.ds(c * CHUNK, CHUNK)], emv[b], se[b])

    def wait_inputs(b):
        pltpu.make_async_copy(xm_hbm.at[idx[b].at[0]], rows[b], sg[b]).wait()
        pltpu.make_async_copy(em_hbm.at[pl.ds(0, CHUNK)], emv[b], se[b]).wait()

    def compute(b):
        rv = rows[b]
        ev = emv[b]

        def row_body(r, _):
            for j in range(D // 16):
                v = rv[r, pl.ds(j * 16, 16)] + ev[r, pl.ds(j * 16, 16)]
                rv[r, pl.ds(j * 16, 16)] = jnp.maximum(v, 0.0)
            return 0

        lax.fori_loop(0, CHUNK, row_body, 0)

    def scatter(b):
        pltpu.async_copy(rows[b], agg_sh.at[idx[b].at[1]], ss[b], add=True)

    def wait_scatter(b):
        pltpu.make_async_copy(rows[b], agg_sh.at[idx[b].at[1]], ss[b]).wait()

    issue(0, 0)

    def pair_body(t, _):
        @pl.when(t > 0)
        def _():
            wait_scatter(1)

        issue(2 * t + 1, 1)
        wait_inputs(0)
        compute(0)
        scatter(0)

        @pl.when(t < NL // 2 - 1)
        def _():
            wait_scatter(0)
            issue(2 * t + 2, 0)

        wait_inputs(1)
        compute(1)
        scatter(1)
        return 0

    lax.fori_loop(0, NL // 2, pair_body, 0)
    wait_scatter(0)
    wait_scatter(1)
    plsc.subcore_barrier()

    pltpu.sync_copy(
        agg_sh.at[pl.ds(sid * ROWS_PER_TILE, ROWS_PER_TILE)],
        out_hbm.at[cid, pl.ds(sid * ROWS_PER_TILE, ROWS_PER_TILE)],
    )


_sc_agg = functools.partial(
    pl.kernel,
    out_type=jax.ShapeDtypeStruct((SC_CORES, N_PAD, D), jnp.float32),
    mesh=plsc.VectorSubcoreMesh(
        core_axis_name="c", subcore_axis_name="s",
        num_cores=SC_CORES, num_subcores=SC_TILES,
    ),
    scratch_types=[
        pltpu.VMEM((2, CHUNK), jnp.int32),
        pltpu.VMEM((2, CHUNK), jnp.int32),
        pltpu.VMEM((CHUNK, D), jnp.float32),
        pltpu.VMEM((CHUNK, D), jnp.float32),
        pltpu.VMEM((CHUNK, D), jnp.float32),
        pltpu.VMEM((CHUNK, D), jnp.float32),
        pltpu.VMEM_SHARED((N_PAD, D), jnp.float32),
        pltpu.SemaphoreType.DMA,
        pltpu.SemaphoreType.DMA,
        pltpu.SemaphoreType.DMA,
        pltpu.SemaphoreType.DMA,
        pltpu.SemaphoreType.DMA,
        pltpu.SemaphoreType.DMA,
    ],
)(_sc_agg_body)


def _upd_body(x_ref, a0_ref, a1_ref, cond_ref, glob_ref,
              wx_ref, wa_ref, wc_ref, wg_ref, b_ref, o_ref):
    acc = jnp.dot(x_ref[...], wx_ref[...], preferred_element_type=jnp.float32)
    agg = a0_ref[...] + a1_ref[...]
    acc += jnp.dot(agg, wa_ref[...], preferred_element_type=jnp.float32)
    acc += cond_ref[...] * wc_ref[...]
    acc += jnp.dot(glob_ref[...], wg_ref[...], preferred_element_type=jnp.float32)
    o_ref[...] = jnp.maximum(acc + b_ref[...], 0.0)


def kernel(x, edge_attr, cond, glob, W_msg, b_msg, W_upd, b_upd, edge_index):
    src = edge_index[0].astype(jnp.int32)
    dst = edge_index[1].astype(jnp.int32)
    npad = E_PAD - E
    src = jnp.concatenate([src, jnp.zeros((npad,), jnp.int32)])
    dst = jnp.concatenate([dst, jnp.full((npad,), N_PAD - 1, jnp.int32)])
    sd = jnp.stack([src.reshape(NCHUNK, CHUNK), dst.reshape(NCHUNK, CHUNK)],
                   axis=1)
    ea = jnp.concatenate([edge_attr, jnp.zeros((npad, DE), jnp.float32)])

    w1 = W_msg[:D]
    w2 = W_msg[D:]
    b_msg2 = b_msg.reshape(1, D)
    wx = W_upd[:D]
    wa = W_upd[D:2 * D]
    wc = W_upd[2 * D:2 * D + NC]
    wg = W_upd[2 * D + NC:]
    b_upd2 = b_upd.reshape(1, D)

    xm = pl.pallas_call(
        _xm_body,
        out_shape=jax.ShapeDtypeStruct((N, D), jnp.float32),
    )(x, w1, b_msg2)

    em = pl.pallas_call(
        _em_body,
        grid=(E_PAD // 4096,),
        in_specs=[
            pl.BlockSpec((4096, DE), lambda i: (i, 0)),
            pl.BlockSpec((DE, D), lambda i: (0, 0)),
        ],
        out_specs=pl.BlockSpec((4096, D), lambda i: (i, 0)),
        out_shape=jax.ShapeDtypeStruct((E_PAD, D), jnp.float32),
    )(ea, w2)

    agg2 = _sc_agg(xm, em, sd)

    out = pl.pallas_call(
        _upd_body,
        out_shape=jax.ShapeDtypeStruct((N, D), jnp.float32),
    )(x, agg2[0, :N], agg2[1, :N], cond, glob, wx, wa, wc, wg, b_upd2)
    return out
